# Initial kernel scaffold; baseline (speedup 1.0000x reference)
import jax
import jax.numpy as jnp
from jax.experimental import pallas as pl


def kernel(x, w, b):
    raise NotImplementedError("write your pallas kernel here")


# speedup vs baseline: 1.1415x; 1.1415x over previous
import jax
import jax.numpy as jnp
from jax.experimental import pallas as pl
from jax.experimental.pallas import tpu as pltpu


def _mm_bias_kernel(x_ref, w_ref, b_ref, o_ref):
    o_ref[...] = (
        jnp.dot(x_ref[...], w_ref[...], preferred_element_type=jnp.float32)
        + b_ref[...]
    ).astype(o_ref.dtype)


def kernel(x, w, b):
    M, K = x.shape
    _, N = w.shape
    out_dtype = x.dtype
    b2 = b.astype(jnp.float32).reshape(1, N)

    tm = 1024
    Mp = -(-M // tm) * tm
    if Mp != M:
        x = jnp.pad(x, ((0, Mp - M), (0, 0)))

    cost = pl.CostEstimate(
        flops=2 * Mp * K * N,
        bytes_accessed=Mp * K * 4 + K * N * 4 + Mp * N * 4 + N * 4,
        transcendentals=0,
    )
    out = pl.pallas_call(
        _mm_bias_kernel,
        out_shape=jax.ShapeDtypeStruct((Mp, N), out_dtype),
        grid=(Mp // tm,),
        in_specs=[
            pl.BlockSpec((tm, K), lambda i: (i, 0)),
            pl.BlockSpec((K, N), lambda i: (0, 0)),
            pl.BlockSpec((1, N), lambda i: (0, 0)),
        ],
        out_specs=pl.BlockSpec((tm, N), lambda i: (i, 0)),
        compiler_params=pltpu.CompilerParams(
            dimension_semantics=("parallel",),
            vmem_limit_bytes=48 << 20,
        ),
        cost_estimate=cost,
    )(x, w, b2)
    if Mp != M:
        out = out[:M]
    return out


# speedup vs baseline: 1.2212x; 1.0698x over previous
import jax
import jax.numpy as jnp
from jax.experimental import pallas as pl
from jax.experimental.pallas import tpu as pltpu


def _mm_bias_kernel(x_ref, w_ref, b_ref, o_ref):
    o_ref[...] = (
        jnp.dot(x_ref[...], w_ref[...], preferred_element_type=jnp.float32)
        + b_ref[...]
    ).astype(o_ref.dtype)


def kernel(x, w, b):
    M, K = x.shape
    _, N = w.shape
    out_dtype = x.dtype
    b2 = b.astype(jnp.float32).reshape(1, N)

    tm = 2048
    Mp = -(-M // tm) * tm
    if Mp != M:
        x = jnp.pad(x, ((0, Mp - M), (0, 0)))

    cost = pl.CostEstimate(
        flops=2 * Mp * K * N,
        bytes_accessed=Mp * K * 4 + K * N * 4 + Mp * N * 4 + N * 4,
        transcendentals=0,
    )
    out = pl.pallas_call(
        _mm_bias_kernel,
        out_shape=jax.ShapeDtypeStruct((Mp, N), out_dtype),
        grid=(Mp // tm,),
        in_specs=[
            pl.BlockSpec((tm, K), lambda i: (i, 0)),
            pl.BlockSpec((K, N), lambda i: (0, 0)),
            pl.BlockSpec((1, N), lambda i: (0, 0)),
        ],
        out_specs=pl.BlockSpec((tm, N), lambda i: (i, 0)),
        compiler_params=pltpu.CompilerParams(
            dimension_semantics=("parallel",),
            vmem_limit_bytes=56 << 20,
        ),
        cost_estimate=cost,
    )(x, w, b2)
    if Mp != M:
        out = out[:M]
    return out
